# baseline (speedup 1.0000x reference)
import functools

import jax
import jax.numpy as jnp
from jax import lax
from jax.experimental import pallas as pl
from jax.experimental.pallas import tpu as pltpu
from jax.experimental.pallas import tpu_sc as plsc

NC, NS = 2, 16
NW = NC * NS
LP = 64


@functools.lru_cache(maxsize=None)
def _sc_call(B, V):
    RPW = B // NW
    mesh = plsc.VectorSubcoreMesh(core_axis_name="c", subcore_axis_name="s")

    @functools.partial(
        pl.kernel,
        out_type=jax.ShapeDtypeStruct((B * V,), jnp.float32),
        mesh=mesh,
        compiler_params=pltpu.CompilerParams(use_tc_tiling_on_sc=False),
        scratch_types=[
            pltpu.VMEM_SHARED((NS, V), jnp.float32),
            pltpu.VMEM((RPW, LP), jnp.int32),
            pltpu.VMEM((RPW, LP), jnp.float32),
            pltpu.VMEM((LP,), jnp.float32),
        ],
    )
    def k(logits_hbm, idx_hbm, maskf_hbm, w_hbm, out_hbm,
          rowsh, idxv, valsv, wv):
        sid = lax.axis_index("s")
        wid = sid * NC + lax.axis_index("c")
        r0 = wid * RPW
        pltpu.sync_copy(idx_hbm.at[pl.ds(r0, RPW)], idxv)
        pltpu.sync_copy(maskf_hbm.at[pl.ds(r0, RPW)], valsv)
        pltpu.sync_copy(w_hbm, wv)

        def val_body(r, _):
            for g in range(LP // 16):
                sl = pl.ds(g * 16, 16)
                valsv[r, sl] = valsv[r, sl] * wv[sl]
            return 0
        lax.fori_loop(0, RPW, val_body, 0)

        def row_body(r, _):
            base = (r0 + r) * V
            pltpu.sync_copy(logits_hbm.at[pl.ds(base, V)], rowsh.at[sid])
            pltpu.sync_copy(valsv.at[r], rowsh.at[sid].at[idxv.at[r]],
                            add=True)
            pltpu.sync_copy(rowsh.at[sid], out_hbm.at[pl.ds(base, V)])
            return 0
        lax.fori_loop(0, RPW, row_body, 0)

    return k


def kernel(logits, loc_seq, mask, decay, boost_scale):
    B, L = loc_seq.shape
    V = logits.shape[1]
    exps = (L - 1 - jnp.arange(L)).astype(jnp.float32)
    weights = (decay.astype(jnp.float32) ** exps) * boost_scale
    w64 = jnp.zeros((LP,), jnp.float32).at[:L].set(weights)
    idx64 = jnp.zeros((B, LP), jnp.int32).at[:, :L].set(loc_seq)
    m64 = jnp.zeros((B, LP), jnp.float32).at[:, :L].set(
        mask.astype(jnp.float32))
    out = _sc_call(B, V)(logits.reshape(-1), idx64, m64, w64)
    return out.reshape(B, V)


# speedup vs baseline: 1.0032x; 1.0032x over previous
import functools

import jax
import jax.numpy as jnp
from jax import lax
from jax.experimental import pallas as pl
from jax.experimental.pallas import tpu as pltpu
from jax.experimental.pallas import tpu_sc as plsc

NC, NS = 2, 16
NW = NC * NS
LP = 64


@functools.lru_cache(maxsize=None)
def _sc_call(B, L, V):
    RPW = B // NW
    NG = (L + 15) // 16
    mesh = plsc.VectorSubcoreMesh(core_axis_name="c", subcore_axis_name="s")

    @functools.partial(
        pl.kernel,
        out_type=jax.ShapeDtypeStruct((B * V,), jnp.float32),
        mesh=mesh,
        compiler_params=pltpu.CompilerParams(needs_layout_passes=False),
        scratch_types=[
            pltpu.VMEM((V,), jnp.float32),
            pltpu.VMEM((RPW * LP,), jnp.int32),
            pltpu.VMEM((RPW * LP,), jnp.float32),
            pltpu.VMEM((LP,), jnp.float32),
        ],
    )
    def k(logits_hbm, idx_hbm, maskf_hbm, w_hbm, out_hbm,
          rowbuf, idxv, valsv, wv):
        sid = lax.axis_index("s")
        wid = sid * NC + lax.axis_index("c")
        r0 = wid * RPW
        pltpu.sync_copy(idx_hbm.at[pl.ds(r0 * LP, RPW * LP)], idxv)
        pltpu.sync_copy(maskf_hbm.at[pl.ds(r0 * LP, RPW * LP)], valsv)
        pltpu.sync_copy(w_hbm, wv)

        def val_body(r, _):
            for g in range(LP // 16):
                sl = pl.ds(r * LP + g * 16, 16)
                valsv[sl] = valsv[sl] * wv[pl.ds(g * 16, 16)]
            return 0
        lax.fori_loop(0, RPW, val_body, 0)

        lanes = lax.iota(jnp.int32, 16)

        def row_body(r, _):
            base = (r0 + r) * V
            pltpu.sync_copy(logits_hbm.at[pl.ds(base, V)], rowbuf)
            for g in range(NG):
                sl = pl.ds(r * LP + g * 16, 16)
                idxg = idxv[sl]
                valg = valsv[sl]
                for lane in range(min(16, L - g * 16)):
                    plsc.addupdate_scatter(rowbuf, [idxg], valg,
                                           mask=lanes == lane)
            pltpu.sync_copy(rowbuf, out_hbm.at[pl.ds(base, V)])
            return 0
        lax.fori_loop(0, RPW, row_body, 0)

    return k


def kernel(logits, loc_seq, mask, decay, boost_scale):
    B, L = loc_seq.shape
    V = logits.shape[1]
    exps = (L - 1 - jnp.arange(L)).astype(jnp.float32)
    weights = (decay.astype(jnp.float32) ** exps) * boost_scale
    w64 = jnp.zeros((LP,), jnp.float32).at[:L].set(weights)
    idx64 = jnp.zeros((B, LP), jnp.int32).at[:, :L].set(loc_seq)
    m64 = jnp.zeros((B, LP), jnp.float32).at[:, :L].set(
        mask.astype(jnp.float32))
    out = _sc_call(B, L, V)(logits.reshape(-1), idx64.reshape(-1),
                            m64.reshape(-1), w64)
    return out.reshape(B, V)


# speedup vs baseline: 1.9826x; 1.9763x over previous
import functools

import jax
import jax.numpy as jnp
from jax import lax
from jax.experimental import pallas as pl
from jax.experimental.pallas import tpu as pltpu
from jax.experimental.pallas import tpu_sc as plsc

NC, NS = 2, 16
NW = NC * NS
LP = 64


@functools.lru_cache(maxsize=None)
def _sc_call(B, L, V):
    RPW = B // NW
    NG = (L + 15) // 16
    mesh = plsc.VectorSubcoreMesh(core_axis_name="c", subcore_axis_name="s")

    @functools.partial(
        pl.kernel,
        out_type=jax.ShapeDtypeStruct((B, V), jnp.float32),
        mesh=mesh,
        compiler_params=pltpu.CompilerParams(needs_layout_passes=False),
        scratch_types=[
            pltpu.VMEM((V,), jnp.float32),
            pltpu.VMEM((RPW * LP,), jnp.int32),
            pltpu.VMEM((RPW * LP,), jnp.float32),
            pltpu.VMEM((LP,), jnp.float32),
        ],
    )
    def k(logits_hbm, idx_hbm, maskf_hbm, w_hbm, out_hbm,
          rowbuf, idxv, valsv, wv):
        sid = lax.axis_index("s")
        wid = sid * NC + lax.axis_index("c")
        r0 = wid * RPW
        pltpu.sync_copy(idx_hbm.at[pl.ds(r0 * LP, RPW * LP)], idxv)
        pltpu.sync_copy(maskf_hbm.at[pl.ds(r0 * LP, RPW * LP)], valsv)
        pltpu.sync_copy(w_hbm, wv)

        def val_body(r, _):
            for g in range(LP // 16):
                sl = pl.ds(r * LP + g * 16, 16)
                valsv[sl] = valsv[sl] * wv[pl.ds(g * 16, 16)]
            return 0
        lax.fori_loop(0, RPW, val_body, 0)

        lanes = lax.iota(jnp.int32, 16)

        def row_body(r, _):
            row = r0 + r
            pltpu.sync_copy(logits_hbm.at[row], rowbuf)
            for g in range(NG):
                sl = pl.ds(r * LP + g * 16, 16)
                idxg = idxv[sl]
                valg = valsv[sl]
                for lane in range(min(16, L - g * 16)):
                    plsc.addupdate_scatter(rowbuf, [idxg], valg,
                                           mask=lanes == lane)
            pltpu.sync_copy(rowbuf, out_hbm.at[row])
            return 0
        lax.fori_loop(0, RPW, row_body, 0)

    return k


def kernel(logits, loc_seq, mask, decay, boost_scale):
    B, L = loc_seq.shape
    V = logits.shape[1]
    exps = (L - 1 - jnp.arange(L)).astype(jnp.float32)
    weights = (decay.astype(jnp.float32) ** exps) * boost_scale
    w64 = jnp.zeros((LP,), jnp.float32).at[:L].set(weights)
    idx64 = jnp.zeros((B, LP), jnp.int32).at[:, :L].set(loc_seq)
    m64 = jnp.zeros((B, LP), jnp.float32).at[:, :L].set(
        mask.astype(jnp.float32))
    return _sc_call(B, L, V)(logits, idx64.reshape(-1),
                             m64.reshape(-1), w64)


# speedup vs baseline: 5.5658x; 2.8073x over previous
import functools

import jax
import jax.numpy as jnp
from jax import lax
from jax.experimental import pallas as pl
from jax.experimental.pallas import tpu as pltpu
from jax.experimental.pallas import tpu_sc as plsc

NC, NS = 2, 16
NW = NC * NS
LP = 64
NQ = 4
LANES = 16


@functools.lru_cache(maxsize=None)
def _sc_call(B, V):
    NB = NW // NQ
    BW = B // NB
    VQ = V // NQ
    VC = 200
    NCH = VQ // VC
    CW = VC * BW
    ROWS = BW * LP
    NGRP = ROWS // LANES
    SVCAP = ROWS + LANES
    mesh = plsc.VectorSubcoreMesh(core_axis_name="c", subcore_axis_name="s")

    @functools.partial(
        pl.kernel,
        out_type=jax.ShapeDtypeStruct((V, B), jnp.float32),
        mesh=mesh,
        compiler_params=pltpu.CompilerParams(needs_layout_passes=False),
        scratch_types=[
            pltpu.VMEM((VC, BW), jnp.float32),
            pltpu.VMEM((VC, BW), jnp.float32),
            pltpu.VMEM((ROWS,), jnp.int32),
            pltpu.VMEM((ROWS,), jnp.float32),
            pltpu.VMEM((SVCAP,), jnp.int32),
            pltpu.VMEM((SVCAP,), jnp.float32),
            pltpu.VMEM((LP,), jnp.float32),
            pltpu.VMEM((128,), jnp.int32),
            pltpu.VMEM((128,), jnp.int32),
            pltpu.SemaphoreType.DMA,
            pltpu.SemaphoreType.DMA,
            pltpu.SemaphoreType.DMA,
            pltpu.SemaphoreType.DMA,
        ],
    )
    def k(lg_hbm, idx_hbm, maskf_hbm, w_hbm, out_hbm,
          buf0, buf1, slab_i, slab_f, sv_off, sv_val, wv, counts, cursors,
          isem0, isem1, osem0, osem1):
        sid = lax.axis_index("s")
        wid = sid * NC + lax.axis_index("c")
        tb = lax.rem(wid, NB)
        q = lax.div(wid, NB)
        v0 = q * VQ
        col0 = pl.multiple_of(tb * BW, BW)
        lane = lax.iota(jnp.int32, LANES)
        lane0 = lane == 0
        ones = jnp.ones((LANES,), jnp.int32)

        bufs = (buf0, buf1)
        isems = (isem0, isem1)
        osems = (osem0, osem1)

        pltpu.sync_copy(idx_hbm.at[pl.ds(tb * ROWS, ROWS)], slab_i)
        pltpu.sync_copy(maskf_hbm.at[pl.ds(tb * ROWS, ROWS)], slab_f)
        pltpu.sync_copy(w_hbm, wv)

        def p1(g, nsv):
            sl = pl.ds(g * LANES, LANES)
            vg = slab_i[sl]
            valg = slab_f[sl] * wv[pl.ds(lax.rem(g, LP // LANES) * LANES,
                                         LANES)]
            b_local = lax.div(g, LP // LANES)
            m = (vg >= v0) & (vg < v0 + VQ)
            foff = (vg - v0) * BW + b_local
            pos = nsv + jnp.cumsum(m.astype(jnp.int32)) - 1
            plsc.store_scatter(sv_off, [pos], foff, mask=m)
            plsc.store_scatter(sv_val, [pos], valg, mask=m)
            return nsv + jnp.max(plsc.all_reduce_population_count(m))
        nsv = lax.fori_loop(0, NGRP, p1, jnp.int32(0))
        ngrp_sv = lax.div(nsv + LANES - 1, LANES)

        for i in range(128 // LANES):
            counts[pl.ds(i * LANES, LANES)] = jnp.zeros((LANES,), jnp.int32)

        def p2(g, _):
            sl = pl.ds(g * LANES, LANES)
            cid = lax.div(sv_off[sl], CW)
            valid = lane < (nsv - g * LANES)
            for l in range(LANES):
                plsc.addupdate_scatter(counts, [cid], ones,
                                       mask=valid & (lane == l))
            return 0
        lax.fori_loop(0, ngrp_sv, p2, 0)

        def p3(i, carry):
            sl = pl.ds(i * LANES, LANES)
            cg = counts[sl]
            cursors[sl] = jnp.cumsum(cg) - cg + carry
            return carry + jnp.sum(cg)
        lax.fori_loop(0, 128 // LANES, p3, jnp.int32(0))

        def p4(g, _):
            sl = pl.ds(g * LANES, LANES)
            off = sv_off[sl]
            val = sv_val[sl]
            cid = lax.div(off, CW)
            local = off - cid * CW
            valid = lane < (nsv - g * LANES)
            for l in range(LANES):
                ml = valid & (lane == l)
                posv = plsc.load_gather(cursors, [cid], mask=ml)
                plsc.store_scatter(slab_i, [posv], local, mask=ml)
                plsc.store_scatter(slab_f, [posv], val, mask=ml)
                plsc.addupdate_scatter(cursors, [cid], ones, mask=ml)
            return 0
        lax.fori_loop(0, ngrp_sv, p4, 0)

        def win_in(c):
            vb = pl.multiple_of(v0 + c * VC, 8)
            return lg_hbm.at[pl.ds(vb, VC), pl.ds(col0, BW)]

        def win_out(c):
            vb = pl.multiple_of(v0 + c * VC, 8)
            return out_hbm.at[pl.ds(vb, VC), pl.ds(col0, BW)]

        def chunk(c, pos, buf, obuf, isem, osem, oisem, oosem):
            pltpu.make_async_copy(win_in(c), buf, isem).wait()
            n_c = jnp.max(plsc.load_gather(
                counts, [jnp.full((LANES,), lax.rem(c, 128), jnp.int32)]))

            def upd(j, _):
                at = jnp.full((LANES,), pos + j, jnp.int32)
                off = plsc.load_gather(slab_i, [at])
                val = plsc.load_gather(slab_f, [at])
                row = lax.div(off, BW)
                col = off - row * BW
                plsc.addupdate_scatter(buf, [row, col], val, mask=lane0)
                return 0
            lax.fori_loop(0, n_c, upd, 0)

            pltpu.async_copy(buf, win_out(c), osem)

            @pl.when(c > 0)
            def _():
                pltpu.make_async_copy(obuf, win_out(c - 1), oosem).wait()

            @pl.when(c + 1 < NCH)
            def _():
                pltpu.async_copy(win_in(c + 1), obuf, oisem)
            return pos + n_c

        pltpu.async_copy(win_in(0), buf0, isem0)

        def pair(i, pos):
            c = i * 2
            pos = chunk(c, pos, buf0, buf1, isem0, osem0, isem1, osem1)
            pos = chunk(c + 1, pos, buf1, buf0, isem1, osem1, isem0, osem0)
            return pos
        pos = lax.fori_loop(0, NCH // 2, pair, jnp.int32(0))

        if NCH % 2 == 1:
            chunk(NCH - 1, pos, buf0, buf1, isem0, osem0, isem1, osem1)
            pltpu.make_async_copy(buf0, win_out(NCH - 1), osem0).wait()
        else:
            pltpu.make_async_copy(buf1, win_out(NCH - 1), osem1).wait()

    return k


def kernel(logits, loc_seq, mask, decay, boost_scale):
    B, L = loc_seq.shape
    V = logits.shape[1]
    exps = (L - 1 - jnp.arange(L)).astype(jnp.float32)
    weights = (decay.astype(jnp.float32) ** exps) * boost_scale
    w64 = jnp.zeros((LP,), jnp.float32).at[:L].set(weights)
    idx64 = jnp.full((B, LP), -1, jnp.int32).at[:, :L].set(loc_seq)
    m64 = jnp.zeros((B, LP), jnp.float32).at[:, :L].set(
        mask.astype(jnp.float32))
    out_t = _sc_call(B, V)(logits.T, idx64.reshape(-1), m64.reshape(-1), w64)
    return out_t.T


# speedup vs baseline: 5.5897x; 1.0043x over previous
import functools

import jax
import jax.numpy as jnp
from jax import lax
from jax.experimental import pallas as pl
from jax.experimental.pallas import tpu as pltpu
from jax.experimental.pallas import tpu_sc as plsc

NC, NS = 2, 16
NW = NC * NS
LP = 64
NQ = 4
LANES = 16


@functools.lru_cache(maxsize=None)
def _sc_call(B, V):
    NB = NW // NQ
    BW = B // NB
    VQ = V // NQ
    VC = 200
    NCH = VQ // VC
    CW = VC * BW
    ROWS = BW * LP
    NGRP = ROWS // LANES
    SVCAP = ROWS + LANES
    mesh = plsc.VectorSubcoreMesh(core_axis_name="c", subcore_axis_name="s")

    @functools.partial(
        pl.kernel,
        out_type=jax.ShapeDtypeStruct((V, B), jnp.float32),
        mesh=mesh,
        compiler_params=pltpu.CompilerParams(needs_layout_passes=False),
        scratch_types=[
            pltpu.VMEM((VC, BW), jnp.float32),
            pltpu.VMEM((VC, BW), jnp.float32),
            pltpu.VMEM((ROWS,), jnp.int32),
            pltpu.VMEM((ROWS,), jnp.float32),
            pltpu.VMEM((SVCAP,), jnp.int32),
            pltpu.VMEM((SVCAP,), jnp.float32),
            pltpu.VMEM((LP,), jnp.float32),
            pltpu.VMEM((128,), jnp.int32),
            pltpu.VMEM((128,), jnp.int32),
            pltpu.SemaphoreType.DMA,
            pltpu.SemaphoreType.DMA,
            pltpu.SemaphoreType.DMA,
            pltpu.SemaphoreType.DMA,
        ],
    )
    def k(lg_hbm, idx_hbm, maskf_hbm, w_hbm, out_hbm,
          buf0, buf1, slab_i, slab_f, sv_off, sv_val, wv, counts, cursors,
          isem0, isem1, osem0, osem1):
        sid = lax.axis_index("s")
        wid = sid * NC + lax.axis_index("c")
        tb = lax.rem(wid, NB)
        q = lax.div(wid, NB)
        v0 = q * VQ
        col0 = pl.multiple_of(tb * BW, BW)
        lane = lax.iota(jnp.int32, LANES)
        lane0 = lane == 0
        ones = jnp.ones((LANES,), jnp.int32)

        def win_in(c):
            vb = pl.multiple_of(v0 + c * VC, 8)
            return lg_hbm.at[pl.ds(vb, VC), pl.ds(col0, BW)]

        def win_out(c):
            vb = pl.multiple_of(v0 + c * VC, 8)
            return out_hbm.at[pl.ds(vb, VC), pl.ds(col0, BW)]

        pltpu.async_copy(win_in(0), buf0, isem0)
        pltpu.async_copy(win_in(1), buf1, isem1)
        pltpu.sync_copy(idx_hbm.at[pl.ds(tb * ROWS, ROWS)], slab_i)
        pltpu.sync_copy(maskf_hbm.at[pl.ds(tb * ROWS, ROWS)], slab_f)
        pltpu.sync_copy(w_hbm, wv)

        def p1(g, nsv):
            sl = pl.ds(g * LANES, LANES)
            vg = slab_i[sl]
            valg = slab_f[sl] * wv[pl.ds(lax.rem(g, LP // LANES) * LANES,
                                         LANES)]
            b_local = lax.div(g, LP // LANES)
            m = (vg >= v0) & (vg < v0 + VQ)
            foff = (vg - v0) * BW + b_local
            pos = nsv + jnp.cumsum(m.astype(jnp.int32)) - 1
            plsc.store_scatter(sv_off, [pos], foff, mask=m)
            plsc.store_scatter(sv_val, [pos], valg, mask=m)
            return nsv + jnp.max(plsc.all_reduce_population_count(m))
        nsv = lax.fori_loop(0, NGRP, p1, jnp.int32(0))
        ngrp_sv = lax.div(nsv + LANES - 1, LANES)

        for i in range(128 // LANES):
            counts[pl.ds(i * LANES, LANES)] = jnp.zeros((LANES,), jnp.int32)

        def p2(g, _):
            sl = pl.ds(g * LANES, LANES)
            cid = lax.div(sv_off[sl], CW)
            valid = lane < (nsv - g * LANES)
            for l in range(LANES):
                plsc.addupdate_scatter(counts, [cid], ones,
                                       mask=valid & (lane == l))
            return 0
        lax.fori_loop(0, ngrp_sv, p2, 0)

        def p3(i, carry):
            sl = pl.ds(i * LANES, LANES)
            cg = counts[sl]
            cursors[sl] = jnp.cumsum(cg) - cg + carry
            return carry + jnp.sum(cg)
        lax.fori_loop(0, 128 // LANES, p3, jnp.int32(0))

        def p4(g, _):
            sl = pl.ds(g * LANES, LANES)
            off = sv_off[sl]
            val = sv_val[sl]
            cid = lax.div(off, CW)
            local = off - cid * CW
            valid = lane < (nsv - g * LANES)
            for l in range(LANES):
                ml = valid & (lane == l)
                posv = plsc.load_gather(cursors, [cid], mask=ml)
                plsc.store_scatter(slab_i, [posv], local, mask=ml)
                plsc.store_scatter(slab_f, [posv], val, mask=ml)
                plsc.addupdate_scatter(cursors, [cid], ones, mask=ml)
            return 0
        lax.fori_loop(0, ngrp_sv, p4, 0)

        def chunk(c, pos, buf, obuf, isem, osem, oisem, oosem):
            pltpu.make_async_copy(win_in(c), buf, isem).wait()
            n_c = jnp.max(plsc.load_gather(
                counts, [jnp.full((LANES,), lax.rem(c, 128), jnp.int32)]))

            def upd(j, _):
                at = jnp.full((LANES,), pos + j, jnp.int32)
                off = plsc.load_gather(slab_i, [at])
                val = plsc.load_gather(slab_f, [at])
                row = lax.div(off, BW)
                col = off - row * BW
                plsc.addupdate_scatter(buf, [row, col], val, mask=lane0)
                return 0
            lax.fori_loop(0, n_c, upd, 0)

            pltpu.async_copy(buf, win_out(c), osem)

            @pl.when(c > 0)
            def _():
                pltpu.make_async_copy(obuf, win_out(c - 1), oosem).wait()

            @pl.when((c > 0) & (c + 1 < NCH))
            def _():
                pltpu.async_copy(win_in(c + 1), obuf, oisem)
            return pos + n_c

        def pair(i, pos):
            c = i * 2
            pos = chunk(c, pos, buf0, buf1, isem0, osem0, isem1, osem1)
            pos = chunk(c + 1, pos, buf1, buf0, isem1, osem1, isem0, osem0)
            return pos
        pos = lax.fori_loop(0, NCH // 2, pair, jnp.int32(0))

        if NCH % 2 == 1:
            chunk(NCH - 1, pos, buf0, buf1, isem0, osem0, isem1, osem1)
            pltpu.make_async_copy(buf0, win_out(NCH - 1), osem0).wait()
        else:
            pltpu.make_async_copy(buf1, win_out(NCH - 1), osem1).wait()

    return k


def kernel(logits, loc_seq, mask, decay, boost_scale):
    B, L = loc_seq.shape
    V = logits.shape[1]
    exps = (L - 1 - jnp.arange(L)).astype(jnp.float32)
    weights = (decay.astype(jnp.float32) ** exps) * boost_scale
    w64 = jnp.zeros((LP,), jnp.float32).at[:L].set(weights)
    idx64 = jnp.full((B, LP), -1, jnp.int32).at[:, :L].set(loc_seq)
    m64 = jnp.zeros((B, LP), jnp.float32).at[:, :L].set(
        mask.astype(jnp.float32))
    out_t = _sc_call(B, V)(logits.T, idx64.reshape(-1), m64.reshape(-1), w64)
    return out_t.T


# speedup vs baseline: 6.0519x; 1.0827x over previous
import functools

import jax
import jax.numpy as jnp
from jax import lax
from jax.experimental import pallas as pl
from jax.experimental.pallas import tpu as pltpu
from jax.experimental.pallas import tpu_sc as plsc

NC, NS = 2, 16
NW = NC * NS
LP = 64
NQ = 4
LANES = 16


@functools.lru_cache(maxsize=None)
def _sc_call(B, L, V):
    NB = NW // NQ
    BW = B // NB
    VQ = V // NQ
    VC = 360
    NCH = -(-VQ // VC)
    VCR = VQ - (NCH - 1) * VC
    CW = VC * BW
    ROWS = BW * LP
    NGRP = ROWS // LANES
    SVCAP = BW * L + LANES
    mesh = plsc.VectorSubcoreMesh(core_axis_name="c", subcore_axis_name="s")

    @functools.partial(
        pl.kernel,
        out_type=jax.ShapeDtypeStruct((V, B), jnp.float32),
        mesh=mesh,
        compiler_params=pltpu.CompilerParams(needs_layout_passes=False),
        scratch_types=[
            pltpu.VMEM((VC, BW), jnp.float32),
            pltpu.VMEM((VC, BW), jnp.float32),
            pltpu.VMEM((ROWS,), jnp.int32),
            pltpu.VMEM((ROWS,), jnp.float32),
            pltpu.VMEM((SVCAP,), jnp.int32),
            pltpu.VMEM((SVCAP,), jnp.float32),
            pltpu.VMEM((LP,), jnp.float32),
            pltpu.VMEM((128,), jnp.int32),
            pltpu.VMEM((128,), jnp.int32),
            pltpu.SemaphoreType.DMA,
            pltpu.SemaphoreType.DMA,
            pltpu.SemaphoreType.DMA,
            pltpu.SemaphoreType.DMA,
        ],
    )
    def k(lg_hbm, idx_hbm, maskf_hbm, w_hbm, out_hbm,
          buf0, buf1, slab_i, slab_f, sv_off, sv_val, wv, counts, cursors,
          isem0, isem1, osem0, osem1):
        sid = lax.axis_index("s")
        wid = sid * NC + lax.axis_index("c")
        tb = lax.rem(wid, NB)
        q = lax.div(wid, NB)
        v0 = q * VQ
        col0 = pl.multiple_of(tb * BW, BW)
        lane = lax.iota(jnp.int32, LANES)
        lane0 = lane == 0
        ones = jnp.ones((LANES,), jnp.int32)

        def win_in(c, n=VC):
            vb = pl.multiple_of(v0 + c * VC, 8)
            return lg_hbm.at[pl.ds(vb, n), pl.ds(col0, BW)]

        def win_out(c, n=VC):
            vb = pl.multiple_of(v0 + c * VC, 8)
            return out_hbm.at[pl.ds(vb, n), pl.ds(col0, BW)]

        pltpu.async_copy(win_in(0), buf0, isem0)
        pltpu.async_copy(win_in(1), buf1, isem1)
        pltpu.sync_copy(idx_hbm.at[pl.ds(tb * ROWS, ROWS)], slab_i)
        pltpu.sync_copy(maskf_hbm.at[pl.ds(tb * ROWS, ROWS)], slab_f)
        pltpu.sync_copy(w_hbm, wv)

        def p1(g, nsv):
            sl = pl.ds(g * LANES, LANES)
            vg = slab_i[sl]
            valg = slab_f[sl] * wv[pl.ds(lax.rem(g, LP // LANES) * LANES,
                                         LANES)]
            b_local = lax.div(g, LP // LANES)
            m = (vg >= v0) & (vg < v0 + VQ)
            foff = (vg - v0) * BW + b_local
            pos = nsv + jnp.cumsum(m.astype(jnp.int32)) - 1
            plsc.store_scatter(sv_off, [pos], foff, mask=m)
            plsc.store_scatter(sv_val, [pos], valg, mask=m)
            return nsv + jnp.max(plsc.all_reduce_population_count(m))
        nsv = lax.fori_loop(0, NGRP, p1, jnp.int32(0))
        ngrp_sv = lax.div(nsv + LANES - 1, LANES)

        for i in range(128 // LANES):
            counts[pl.ds(i * LANES, LANES)] = jnp.zeros((LANES,), jnp.int32)

        def p2(g, _):
            sl = pl.ds(g * LANES, LANES)
            cid = lax.div(sv_off[sl], CW)
            valid = lane < (nsv - g * LANES)
            for l in range(LANES):
                plsc.addupdate_scatter(counts, [cid], ones,
                                       mask=valid & (lane == l))
            return 0
        lax.fori_loop(0, ngrp_sv, p2, 0)

        def p3(i, carry):
            sl = pl.ds(i * LANES, LANES)
            cg = counts[sl]
            cursors[sl] = jnp.cumsum(cg) - cg + carry
            return carry + jnp.sum(cg)
        lax.fori_loop(0, 128 // LANES, p3, jnp.int32(0))

        def p4(g, _):
            sl = pl.ds(g * LANES, LANES)
            off = sv_off[sl]
            val = sv_val[sl]
            cid = lax.div(off, CW)
            local = off - cid * CW
            valid = lane < (nsv - g * LANES)
            for l in range(LANES):
                ml = valid & (lane == l)
                posv = plsc.load_gather(cursors, [cid], mask=ml)
                plsc.store_scatter(slab_i, [posv], local, mask=ml)
                plsc.store_scatter(slab_f, [posv], val, mask=ml)
                plsc.addupdate_scatter(cursors, [cid], ones, mask=ml)
            return 0
        lax.fori_loop(0, ngrp_sv, p4, 0)

        def chunk(c, pos, buf, obuf, isem, osem, oisem, oosem,
                  issue_next=True, n=VC):
            bufn = buf.at[pl.ds(0, n)]
            pltpu.make_async_copy(win_in(c, n), bufn, isem).wait()
            n_c = jnp.max(plsc.load_gather(
                counts, [jnp.full((LANES,), lax.rem(c, 128), jnp.int32)]))

            def upd(j, _):
                at = jnp.full((LANES,), pos + j, jnp.int32)
                off = plsc.load_gather(slab_i, [at])
                val = plsc.load_gather(slab_f, [at])
                row = lax.div(off, BW)
                col = off - row * BW
                plsc.addupdate_scatter(buf, [row, col], val, mask=lane0)
                return 0
            lax.fori_loop(0, n_c, upd, 0)

            pltpu.async_copy(bufn, win_out(c, n), osem)

            @pl.when(c > 0)
            def _():
                pltpu.make_async_copy(obuf, win_out(c - 1), oosem).wait()

            if issue_next:
                @pl.when(c > 0)
                def _():
                    pltpu.async_copy(win_in(c + 1), obuf, oisem)
            return pos + n_c

        assert NCH % 2 == 0
        def pair(i, pos):
            c = i * 2
            pos = chunk(c, pos, buf0, buf1, isem0, osem0, isem1, osem1)
            pos = chunk(c + 1, pos, buf1, buf0, isem1, osem1, isem0, osem0)
            return pos
        pos = lax.fori_loop(0, NCH // 2 - 1, pair, jnp.int32(0))

        c = NCH - 2
        pos = chunk(c, pos, buf0, buf1, isem0, osem0, isem1, osem1,
                    issue_next=False)
        pltpu.async_copy(win_in(NCH - 1, VCR), buf1.at[pl.ds(0, VCR)], isem1)
        pos = chunk(NCH - 1, pos, buf1, buf0, isem1, osem1, isem0, osem0,
                    issue_next=False, n=VCR)
        pltpu.make_async_copy(buf1.at[pl.ds(0, VCR)], win_out(NCH - 1, VCR),
                              osem1).wait()

    return k


def kernel(logits, loc_seq, mask, decay, boost_scale):
    B, L = loc_seq.shape
    V = logits.shape[1]
    exps = (L - 1 - jnp.arange(L)).astype(jnp.float32)
    weights = (decay.astype(jnp.float32) ** exps) * boost_scale
    w64 = jnp.zeros((LP,), jnp.float32).at[:L].set(weights)
    idx64 = jnp.full((B, LP), -1, jnp.int32).at[:, :L].set(loc_seq)
    m64 = jnp.zeros((B, LP), jnp.float32).at[:, :L].set(
        mask.astype(jnp.float32))
    out_t = _sc_call(B, L, V)(logits.T, idx64.reshape(-1),
                              m64.reshape(-1), w64)
    return out_t.T


# speedup vs baseline: 6.0871x; 1.0058x over previous
import functools

import jax
import jax.numpy as jnp
from jax import lax
from jax.experimental import pallas as pl
from jax.experimental.pallas import tpu as pltpu
from jax.experimental.pallas import tpu_sc as plsc

NC, NS = 2, 16
NW = NC * NS
LP = 64
NQ = 4
LANES = 16


@functools.lru_cache(maxsize=None)
def _sc_call(B, L, V):
    NB = NW // NQ
    BW = B // NB
    VQ = V // NQ
    VC = 256
    NCH = -(-VQ // VC)
    VCR = VQ - (NCH - 1) * VC
    CW = VC * BW
    ROWS = BW * LP
    NGRP = ROWS // LANES
    SVCAP = BW * L + LANES
    assert (NCH - 2) % 3 == 0 and VC % 8 == 0 and VCR % 8 == 0
    mesh = plsc.VectorSubcoreMesh(core_axis_name="c", subcore_axis_name="s")

    @functools.partial(
        pl.kernel,
        out_type=jax.ShapeDtypeStruct((V, B), jnp.float32),
        mesh=mesh,
        compiler_params=pltpu.CompilerParams(needs_layout_passes=False),
        scratch_types=[
            pltpu.VMEM((VC, BW), jnp.float32),
            pltpu.VMEM((VC, BW), jnp.float32),
            pltpu.VMEM((VC, BW), jnp.float32),
            pltpu.VMEM((ROWS,), jnp.int32),
            pltpu.VMEM((ROWS,), jnp.float32),
            pltpu.VMEM((SVCAP,), jnp.int32),
            pltpu.VMEM((SVCAP,), jnp.float32),
            pltpu.VMEM((LP,), jnp.float32),
            pltpu.VMEM((128,), jnp.int32),
            pltpu.VMEM((128,), jnp.int32),
            pltpu.SemaphoreType.DMA,
            pltpu.SemaphoreType.DMA,
            pltpu.SemaphoreType.DMA,
            pltpu.SemaphoreType.DMA,
            pltpu.SemaphoreType.DMA,
            pltpu.SemaphoreType.DMA,
        ],
    )
    def k(lg_hbm, idx_hbm, maskf_hbm, w_hbm, out_hbm,
          buf0, buf1, buf2, slab_i, slab_f, sv_off, sv_val, wv,
          counts, cursors, isem0, isem1, isem2, osem0, osem1, osem2):
        sid = lax.axis_index("s")
        wid = sid * NC + lax.axis_index("c")
        tb = lax.rem(wid, NB)
        q = lax.div(wid, NB)
        v0 = q * VQ
        col0 = pl.multiple_of(tb * BW, BW)
        lane = lax.iota(jnp.int32, LANES)
        lane0 = lane == 0
        ones = jnp.ones((LANES,), jnp.int32)

        def win_in(c, n=VC):
            vb = pl.multiple_of(v0 + c * VC, 8)
            return lg_hbm.at[pl.ds(vb, n), pl.ds(col0, BW)]

        def win_out(c, n=VC):
            vb = pl.multiple_of(v0 + c * VC, 8)
            return out_hbm.at[pl.ds(vb, n), pl.ds(col0, BW)]

        pltpu.async_copy(win_in(0), buf0, isem0)
        pltpu.async_copy(win_in(1), buf1, isem1)
        pltpu.async_copy(win_in(2), buf2, isem2)
        pltpu.sync_copy(idx_hbm.at[pl.ds(tb * ROWS, ROWS)], slab_i)
        pltpu.sync_copy(maskf_hbm.at[pl.ds(tb * ROWS, ROWS)], slab_f)
        pltpu.sync_copy(w_hbm, wv)

        def p1(g, nsv):
            sl = pl.ds(g * LANES, LANES)
            vg = slab_i[sl]
            valg = slab_f[sl] * wv[pl.ds(lax.rem(g, LP // LANES) * LANES,
                                         LANES)]
            b_local = lax.div(g, LP // LANES)
            m = (vg >= v0) & (vg < v0 + VQ)
            foff = (vg - v0) * BW + b_local
            pos = nsv + jnp.cumsum(m.astype(jnp.int32)) - 1
            plsc.store_scatter(sv_off, [pos], foff, mask=m)
            plsc.store_scatter(sv_val, [pos], valg, mask=m)
            return nsv + jnp.max(plsc.all_reduce_population_count(m))
        nsv = lax.fori_loop(0, NGRP, p1, jnp.int32(0))
        ngrp_sv = lax.div(nsv + LANES - 1, LANES)

        for i in range(128 // LANES):
            counts[pl.ds(i * LANES, LANES)] = jnp.zeros((LANES,), jnp.int32)

        def p2(g, _):
            sl = pl.ds(g * LANES, LANES)
            cid = lax.div(sv_off[sl], CW)
            valid = lane < (nsv - g * LANES)
            for l in range(LANES):
                plsc.addupdate_scatter(counts, [cid], ones,
                                       mask=valid & (lane == l))
            return 0
        lax.fori_loop(0, ngrp_sv, p2, 0)

        def p3(i, carry):
            sl = pl.ds(i * LANES, LANES)
            cg = counts[sl]
            cursors[sl] = jnp.cumsum(cg) - cg + carry
            return carry + jnp.sum(cg)
        lax.fori_loop(0, 128 // LANES, p3, jnp.int32(0))

        def p4(g, _):
            sl = pl.ds(g * LANES, LANES)
            off = sv_off[sl]
            val = sv_val[sl]
            cid = lax.div(off, CW)
            local = off - cid * CW
            valid = lane < (nsv - g * LANES)
            for l in range(LANES):
                ml = valid & (lane == l)
                posv = plsc.load_gather(cursors, [cid], mask=ml)
                plsc.store_scatter(slab_i, [posv], local, mask=ml)
                plsc.store_scatter(slab_f, [posv], val, mask=ml)
                plsc.addupdate_scatter(cursors, [cid], ones, mask=ml)
            return 0
        lax.fori_loop(0, ngrp_sv, p4, 0)

        def chunk(c, pos, buf, nbuf, isem, osem, nisem, nosem,
                  issue_next=True, n=VC):
            bufn = buf.at[pl.ds(0, n)]
            pltpu.make_async_copy(win_in(c, n), bufn, isem).wait()
            n_c = jnp.max(plsc.load_gather(
                counts, [jnp.full((LANES,), lax.rem(c, 128), jnp.int32)]))

            def upd(j, _):
                at = jnp.full((LANES,), pos + j, jnp.int32)
                off = plsc.load_gather(slab_i, [at])
                val = plsc.load_gather(slab_f, [at])
                row = lax.div(off, BW)
                col = off - row * BW
                plsc.addupdate_scatter(buf, [row, col], val, mask=lane0)
                return 0
            lax.fori_loop(0, n_c, upd, 0)

            pltpu.async_copy(bufn, win_out(c, n), osem)

            @pl.when(c >= 2)
            def _():
                pltpu.make_async_copy(nbuf, win_out(c - 2), nosem).wait()

            if issue_next:
                @pl.when((c >= 2) & (c + 1 < NCH - 1))
                def _():
                    pltpu.async_copy(win_in(c + 1), nbuf, nisem)
            return pos + n_c

        def triplet(i, pos):
            c = i * 3
            pos = chunk(c, pos, buf0, buf1, isem0, osem0, isem1, osem1)
            pos = chunk(c + 1, pos, buf1, buf2, isem1, osem1, isem2, osem2)
            pos = chunk(c + 2, pos, buf2, buf0, isem2, osem2, isem0, osem0)
            return pos
        pos = lax.fori_loop(0, (NCH - 2) // 3, triplet, jnp.int32(0))

        c = NCH - 2
        pos = chunk(c, pos, buf0, buf1, isem0, osem0, isem1, osem1,
                    issue_next=False)
        pltpu.async_copy(win_in(NCH - 1, VCR), buf1.at[pl.ds(0, VCR)], isem1)
        pos = chunk(NCH - 1, pos, buf1, buf2, isem1, osem1, isem2, osem2,
                    issue_next=False, n=VCR)
        pltpu.make_async_copy(buf0, win_out(NCH - 2), osem0).wait()
        pltpu.make_async_copy(buf1.at[pl.ds(0, VCR)], win_out(NCH - 1, VCR),
                              osem1).wait()

    return k


def kernel(logits, loc_seq, mask, decay, boost_scale):
    B, L = loc_seq.shape
    V = logits.shape[1]
    exps = (L - 1 - jnp.arange(L)).astype(jnp.float32)
    weights = (decay.astype(jnp.float32) ** exps) * boost_scale
    w64 = jnp.zeros((LP,), jnp.float32).at[:L].set(weights)
    idx64 = jnp.full((B, LP), -1, jnp.int32).at[:, :L].set(loc_seq)
    m64 = jnp.zeros((B, LP), jnp.float32).at[:, :L].set(
        mask.astype(jnp.float32))
    out_t = _sc_call(B, L, V)(logits.T, idx64.reshape(-1),
                              m64.reshape(-1), w64)
    return out_t.T
